# Initial kernel scaffold; baseline (speedup 1.0000x reference)
import jax
import jax.numpy as jnp
from jax.experimental import pallas as pl


def kernel(x, flow, metric):
    raise NotImplementedError("write your pallas kernel here")


# speedup vs baseline: 2.0854x; 2.0854x over previous
import functools

import jax
import jax.numpy as jnp
from jax import lax
from jax.experimental import pallas as pl
from jax.experimental.pallas import tpu as pltpu
from jax.experimental.pallas import tpu_sc as plsc

NC, NS, L = 2, 16, 16


def _prep_body(flow_ref, metric_ref, packed_ref, tx_ref, wn_ref, ws_ref,
               *, Hb, H, W):
    h = pl.program_id(1)
    rows = (h * Hb + lax.broadcasted_iota(jnp.int32, (Hb, W), 0)).astype(
        jnp.float32)
    cols = lax.broadcasted_iota(jnp.int32, (Hb, W), 1).astype(jnp.float32)
    fx = cols + flow_ref[0, 0]
    fy = rows + flow_ref[0, 1]
    x0f = jnp.floor(fx)
    y0f = jnp.floor(fy)
    x1f = x0f + 1.0
    y1f = y0f + 1.0
    xi0 = x0f.astype(jnp.int32)
    yi0 = y0f.astype(jnp.int32)
    xi1 = x1f.astype(jnp.int32)
    yi1 = y1f.astype(jnp.int32)
    cx0 = jnp.clip(xi0, 0, W - 1)
    cx1 = jnp.clip(xi1, 0, W - 1)
    cy0 = jnp.clip(yi0, 0, H - 1)
    cy1 = jnp.clip(yi1, 0, H - 1)
    expm = jnp.exp(metric_ref[0, 0])

    def vx(xi):
        return (xi >= 0) & (xi < W)

    def vy(yi):
        return (yi >= 0) & (yi < H)

    packed_ref[0] = (
        (cy0 * W + cx0) * 4 + (cy1 - cy0) * 2 + (cx1 - cx0)
        + (vx(xi0).astype(jnp.int32) << 20)
        + (vx(xi1).astype(jnp.int32) << 21)
    )
    zero = jnp.zeros_like(fx)
    tx_ref[0] = fx - x0f
    wn_ref[0] = jnp.where(vy(yi0), (y1f - fy) * expm, zero)
    ws_ref[0] = jnp.where(vy(yi1), (fy - y0f) * expm, zero)


def _prep(flow, metric, *, B, H, W, Hb=64):
    io_spec = pl.BlockSpec((1, Hb, W), lambda b, h: (b, h, 0))
    return pl.pallas_call(
        functools.partial(_prep_body, Hb=Hb, H=H, W=W),
        grid=(B, H // Hb),
        in_specs=[
            pl.BlockSpec((1, 2, Hb, W), lambda b, h: (b, 0, h, 0)),
            pl.BlockSpec((1, 1, Hb, W), lambda b, h: (b, 0, h, 0)),
        ],
        out_specs=[io_spec] * 4,
        out_shape=[jax.ShapeDtypeStruct((B, H, W), jnp.int32)]
        + [jax.ShapeDtypeStruct((B, H, W), jnp.float32)] * 3,
    )(flow, metric)


def _splat(x_flat, packed, tx, wn, ws, *, B, C, H, W):
    HW = H * W
    PPT = HW // NS
    WIN = 1024
    NWIN = PPT // WIN
    ZB = 1024
    RPT = PPT // W
    mesh = plsc.VectorSubcoreMesh(
        core_axis_name="c", subcore_axis_name="s", num_cores=NC, num_subcores=NS
    )

    @functools.partial(
        pl.kernel,
        out_type=jax.ShapeDtypeStruct((B, C, H, W), jnp.float32),
        mesh=mesh,
        scratch_types=[
            pltpu.VMEM((PPT,), jnp.int32),
            pltpu.VMEM((PPT,), jnp.float32),
            pltpu.VMEM((PPT,), jnp.float32),
            pltpu.VMEM((PPT,), jnp.float32),
            pltpu.VMEM((PPT // 4,), jnp.float32),
            pltpu.VMEM((PPT,), jnp.float32),
            pltpu.VMEM((RPT // 2, W), jnp.float32),
            [[pltpu.VMEM((WIN,), jnp.int32) for _ in range(4)]
             for _ in range(2)],
            [[pltpu.VMEM((WIN,), jnp.float32) for _ in range(4)]
             for _ in range(2)],
            pltpu.VMEM((ZB,), jnp.float32),
            pltpu.VMEM_SHARED((HW,), jnp.float32),
            [pltpu.SemaphoreType.DMA for _ in range(2)],
        ],
    )
    def splat_kernel(x_hbm, pk_hbm, tx_hbm, wn_hbm, ws_hbm, out_hbm,
                     pk_v, tx_v, wn_v, ws_v, x_v, den_v, norm_v, idx4_v,
                     upd4_v, zero_v, plane_sh, ssem):
        sc = lax.axis_index("c")
        sid = lax.axis_index("s")
        off = sc * HW + sid * PPT
        q0 = sid * PPT
        XCH = PPT // 4
        WPH = XCH // WIN
        HALF = PPT // 2

        pltpu.sync_copy(pk_hbm.at[pl.ds(off, PPT)], pk_v)
        pltpu.sync_copy(tx_hbm.at[pl.ds(off, PPT)], tx_v)
        pltpu.sync_copy(wn_hbm.at[pl.ds(off, PPT)], wn_v)
        pltpu.sync_copy(ws_hbm.at[pl.ds(off, PPT)], ws_v)

        def zbody(i, _):
            zero_v[pl.ds(i * L, L)] = jnp.zeros((L,), jnp.float32)
            return 0

        lax.fori_loop(0, ZB // L, zbody, 0)

        def zero_my_slice():
            for r in range(PPT // ZB):
                pltpu.sync_copy(zero_v, plane_sh.at[pl.ds(q0 + r * ZB, ZB)])

        zero_my_slice()
        plsc.subcore_barrier()

        def do_windows(with_x, c):
            pend = [None, None]
            for wnd in range(NWIN):
                bi = wnd % 2
                idxb, updb = idx4_v[bi], upd4_v[bi]
                if pend[bi] is not None:
                    for d in pend[bi]:
                        d.wait()

                def body(i, _, wnd=wnd, idxb=idxb, updb=updb):
                    j = wnd * (WIN // L) + i
                    s = pl.ds(j * L, L)
                    pk = pk_v[s]
                    base = (pk >> 2) & 0x3FFFF
                    dxb = pk & 1
                    dyw = (pk & 2) << 8
                    vx0 = (pk & (1 << 20)) != 0
                    vx1 = (pk & (1 << 21)) != 0
                    tx = tx_v[s]
                    wn = wn_v[s]
                    ws = ws_v[s]
                    omt = 1.0 - tx
                    zero = jnp.zeros((L,), jnp.float32)
                    w4 = (
                        jnp.where(vx0, omt * wn, zero),
                        jnp.where(vx1, tx * wn, zero),
                        jnp.where(vx0, omt * ws, zero),
                        jnp.where(vx1, tx * ws, zero),
                    )
                    if with_x:
                        xv = x_v[pl.ds((wnd % WPH) * WIN + i * L, L)]
                    else:
                        xv = None
                    sw = pl.ds(i * L, L)
                    for k, idx in enumerate(
                            (base, base + dxb, base + dyw, base + dxb + dyw)):
                        idxb[k][sw] = idx
                        updb[k][sw] = (xv * w4[k]) if with_x else w4[k]
                    return 0

                lax.fori_loop(0, WIN // L, body, 0)
                pend[bi] = [
                    pltpu.async_copy(
                        updb[k], plane_sh.at[idxb[k]], ssem[bi], add=True
                    )
                    for k in range(4)
                ]
                if with_x and wnd % WPH == WPH - 1 and wnd != NWIN - 1:
                    nxt = (wnd // WPH + 1) * XCH
                    pltpu.sync_copy(
                        x_hbm.at[pl.ds((sc * C + c) * HW + q0 + nxt, XCH)],
                        x_v)
            for p in pend:
                if p is not None:
                    for d in p:
                        d.wait()

        do_windows(False, 0)
        plsc.subcore_barrier()
        pltpu.sync_copy(plane_sh.at[pl.ds(q0, PPT)], den_v)
        zero_my_slice()
        plsc.subcore_barrier()

        NR = RPT // 2

        def chan_body(c, _):
            pltpu.sync_copy(x_hbm.at[pl.ds((sc * C + c) * HW + q0, XCH)], x_v)
            do_windows(True, c)
            plsc.subcore_barrier()
            for half in range(2):
                hq = half * HALF
                for r in range(NR):
                    pltpu.sync_copy(
                        plane_sh.at[pl.ds(q0 + hq + r * W, W)], norm_v.at[r])

                def nbody(i, _, hq=hq):
                    rr = i // (W // L)
                    cc = (i % (W // L)) * L
                    norm_v[rr, pl.ds(cc, L)] = norm_v[rr, pl.ds(cc, L)] / (
                        den_v[pl.ds(hq + i * L, L)] + 1e-7)
                    return 0

                lax.fori_loop(0, HALF // L, nbody, 0)
                pltpu.sync_copy(
                    norm_v,
                    out_hbm.at[sc, c, pl.ds(sid * RPT + half * NR, NR)])
            zero_my_slice()
            plsc.subcore_barrier()
            return 0

        lax.fori_loop(0, C, chan_body, 0)

    return splat_kernel(x_flat, packed, tx, wn, ws)


def kernel(x, flow, metric):
    B, C, H, W = x.shape
    HW = H * W
    N = B * HW
    pk, tx, wn, ws = _prep(flow, metric, B=B, H=H, W=W)
    return _splat(
        x.reshape(B * C * HW),
        pk.reshape(N), tx.reshape(N), wn.reshape(N), ws.reshape(N),
        B=B, C=C, H=H, W=W,
    )


# speedup vs baseline: 2.3341x; 1.1193x over previous
import functools

import jax
import jax.numpy as jnp
from jax import lax
from jax.experimental import pallas as pl
from jax.experimental.pallas import tpu as pltpu
from jax.experimental.pallas import tpu_sc as plsc

NC, NS, L = 2, 16, 16


def _prep_body(flow_ref, metric_ref, packed_ref, tx_ref, wn_ref, ws_ref,
               *, Hb, H, W):
    h = pl.program_id(1)
    rows = (h * Hb + lax.broadcasted_iota(jnp.int32, (Hb, W), 0)).astype(
        jnp.float32)
    cols = lax.broadcasted_iota(jnp.int32, (Hb, W), 1).astype(jnp.float32)
    fx = cols + flow_ref[0, 0]
    fy = rows + flow_ref[0, 1]
    x0f = jnp.floor(fx)
    y0f = jnp.floor(fy)
    x1f = x0f + 1.0
    y1f = y0f + 1.0
    xi0 = x0f.astype(jnp.int32)
    yi0 = y0f.astype(jnp.int32)
    xi1 = x1f.astype(jnp.int32)
    yi1 = y1f.astype(jnp.int32)
    cx0 = jnp.clip(xi0, 0, W - 1)
    cx1 = jnp.clip(xi1, 0, W - 1)
    cy0 = jnp.clip(yi0, 0, H - 1)
    cy1 = jnp.clip(yi1, 0, H - 1)
    expm = jnp.exp(metric_ref[0, 0])

    def vx(xi):
        return (xi >= 0) & (xi < W)

    def vy(yi):
        return (yi >= 0) & (yi < H)

    packed_ref[0] = (
        (cy0 * W + cx0) * 4 + (cy1 - cy0) * 2 + (cx1 - cx0)
        + (vx(xi0).astype(jnp.int32) << 20)
        + (vx(xi1).astype(jnp.int32) << 21)
    )
    zero = jnp.zeros_like(fx)
    tx_ref[0] = fx - x0f
    wn_ref[0] = jnp.where(vy(yi0), (y1f - fy) * expm, zero)
    ws_ref[0] = jnp.where(vy(yi1), (fy - y0f) * expm, zero)


def _prep(flow, metric, *, B, H, W, Hb=64):
    io_spec = pl.BlockSpec((1, Hb, W), lambda b, h: (b, h, 0))
    return pl.pallas_call(
        functools.partial(_prep_body, Hb=Hb, H=H, W=W),
        grid=(B, H // Hb),
        in_specs=[
            pl.BlockSpec((1, 2, Hb, W), lambda b, h: (b, 0, h, 0)),
            pl.BlockSpec((1, 1, Hb, W), lambda b, h: (b, 0, h, 0)),
        ],
        out_specs=[io_spec] * 4,
        out_shape=[jax.ShapeDtypeStruct((B, H, W), jnp.int32)]
        + [jax.ShapeDtypeStruct((B, H, W), jnp.float32)] * 3,
    )(flow, metric)


def _splat(x_flat, packed, tx, wn, ws, *, B, C, HW, W):
    PPT = HW // NS
    WIN = 1024
    NWIN = PPT // WIN
    ZB = 2048
    mesh = plsc.VectorSubcoreMesh(
        core_axis_name="c", subcore_axis_name="s", num_cores=NC, num_subcores=NS
    )

    @functools.partial(
        pl.kernel,
        out_type=jax.ShapeDtypeStruct((B * C * HW,), jnp.float32),
        mesh=mesh,
        scratch_types=[
            pltpu.VMEM((PPT,), jnp.int32),
            pltpu.VMEM((PPT,), jnp.float32),
            pltpu.VMEM((PPT,), jnp.float32),
            pltpu.VMEM((PPT,), jnp.float32),
            pltpu.VMEM((PPT // 2,), jnp.float32),
            pltpu.VMEM((PPT,), jnp.float32),
            [[pltpu.VMEM((WIN,), jnp.int32) for _ in range(4)]
             for _ in range(2)],
            [[pltpu.VMEM((WIN,), jnp.float32) for _ in range(4)]
             for _ in range(2)],
            pltpu.VMEM((ZB,), jnp.float32),
            pltpu.VMEM_SHARED((HW,), jnp.float32),
            [pltpu.SemaphoreType.DMA for _ in range(2)],
        ],
    )
    def splat_kernel(x_hbm, pk_hbm, tx_hbm, wn_hbm, ws_hbm, out_hbm,
                     pk_v, tx_v, wn_v, ws_v, x_v, den_v, idx4_v, upd4_v,
                     zero_v, plane_sh, ssem):
        sc = lax.axis_index("c")
        sid = lax.axis_index("s")
        off = sc * HW + sid * PPT
        q0 = sid * PPT
        HALF = PPT // 2
        WPH = HALF // WIN

        pltpu.sync_copy(pk_hbm.at[pl.ds(off, PPT)], pk_v)
        pltpu.sync_copy(tx_hbm.at[pl.ds(off, PPT)], tx_v)
        pltpu.sync_copy(wn_hbm.at[pl.ds(off, PPT)], wn_v)
        pltpu.sync_copy(ws_hbm.at[pl.ds(off, PPT)], ws_v)

        def zbody(i, _):
            zero_v[pl.ds(i * L, L)] = jnp.zeros((L,), jnp.float32)
            return 0

        lax.fori_loop(0, ZB // L, zbody, 0)

        def zero_my_slice():
            for r in range(PPT // ZB):
                pltpu.sync_copy(zero_v, plane_sh.at[pl.ds(q0 + r * ZB, ZB)])

        zero_my_slice()
        plsc.subcore_barrier()

        def do_windows(with_x, c):
            pend = [None, None]
            for wnd in range(NWIN):
                bi = wnd % 2
                idxb, updb = idx4_v[bi], upd4_v[bi]
                if pend[bi] is not None:
                    for d in pend[bi]:
                        d.wait()

                def body(i, _, wnd=wnd, idxb=idxb, updb=updb):
                    j = wnd * (WIN // L) + i
                    s = pl.ds(j * L, L)
                    pk = pk_v[s]
                    base = (pk >> 2) & 0x3FFFF
                    dxb = pk & 1
                    dyw = (pk & 2) << 8
                    vx0 = (pk & (1 << 20)) != 0
                    vx1 = (pk & (1 << 21)) != 0
                    tx = tx_v[s]
                    wn = wn_v[s]
                    ws = ws_v[s]
                    omt = 1.0 - tx
                    zero = jnp.zeros((L,), jnp.float32)
                    w4 = (
                        jnp.where(vx0, omt * wn, zero),
                        jnp.where(vx1, tx * wn, zero),
                        jnp.where(vx0, omt * ws, zero),
                        jnp.where(vx1, tx * ws, zero),
                    )
                    if with_x:
                        xv = x_v[pl.ds((wnd % WPH) * WIN + i * L, L)]
                    sw = pl.ds(i * L, L)
                    for k, idx in enumerate(
                            (base, base + dxb, base + dyw, base + dxb + dyw)):
                        idxb[k][sw] = idx
                        updb[k][sw] = (xv * w4[k]) if with_x else w4[k]
                    return 0

                lax.fori_loop(0, WIN // L, body, 0)
                pend[bi] = [
                    pltpu.async_copy(
                        updb[k], plane_sh.at[idxb[k]], ssem[bi], add=True
                    )
                    for k in range(4)
                ]
                if with_x and wnd == WPH - 1:
                    pltpu.sync_copy(
                        x_hbm.at[pl.ds((sc * C + c) * HW + q0 + HALF, HALF)],
                        x_v)
            for p in pend:
                if p is not None:
                    for d in p:
                        d.wait()

        do_windows(False, 0)
        plsc.subcore_barrier()
        pltpu.sync_copy(plane_sh.at[pl.ds(q0, PPT)], den_v)
        zero_my_slice()
        plsc.subcore_barrier()

        def chan_body(c, _):
            pltpu.sync_copy(x_hbm.at[pl.ds((sc * C + c) * HW + q0, HALF)], x_v)
            do_windows(True, c)
            plsc.subcore_barrier()
            for half in range(2):
                hq = half * HALF
                pltpu.sync_copy(plane_sh.at[pl.ds(q0 + hq, HALF)], x_v)

                def nbody(i, _, hq=hq):
                    s = pl.ds(i * L, L)
                    x_v[s] = x_v[s] / (den_v[pl.ds(hq + i * L, L)] + 1e-7)
                    return 0

                lax.fori_loop(0, HALF // L, nbody, 0)
                pltpu.sync_copy(
                    x_v,
                    out_hbm.at[pl.ds((sc * C + c) * HW + q0 + hq, HALF)])
            zero_my_slice()
            plsc.subcore_barrier()
            return 0

        lax.fori_loop(0, C, chan_body, 0)

    return splat_kernel(x_flat, packed, tx, wn, ws)


def kernel(x, flow, metric):
    B, C, H, W = x.shape
    HW = H * W
    N = B * HW
    pk, tx, wn, ws = _prep(flow, metric, B=B, H=H, W=W)
    out = _splat(
        x.reshape(B * C * HW),
        pk.reshape(N), tx.reshape(N), wn.reshape(N), ws.reshape(N),
        B=B, C=C, HW=HW, W=W,
    )
    return out.reshape(B, C, H, W)


# speedup vs baseline: 2.3688x; 1.0149x over previous
import functools

import jax
import jax.numpy as jnp
from jax import lax
from jax.experimental import pallas as pl
from jax.experimental.pallas import tpu as pltpu
from jax.experimental.pallas import tpu_sc as plsc

NC, NS, L = 2, 16, 16


def _prep_body(flow_ref, metric_ref, packed_ref, tx_ref, wn_ref, ws_ref,
               *, Hb, H, W):
    h = pl.program_id(1)
    rows = (h * Hb + lax.broadcasted_iota(jnp.int32, (Hb, W), 0)).astype(
        jnp.float32)
    cols = lax.broadcasted_iota(jnp.int32, (Hb, W), 1).astype(jnp.float32)
    fx = cols + flow_ref[0, 0]
    fy = rows + flow_ref[0, 1]
    x0f = jnp.floor(fx)
    y0f = jnp.floor(fy)
    x1f = x0f + 1.0
    y1f = y0f + 1.0
    xi0 = x0f.astype(jnp.int32)
    yi0 = y0f.astype(jnp.int32)
    xi1 = x1f.astype(jnp.int32)
    yi1 = y1f.astype(jnp.int32)
    cx0 = jnp.clip(xi0, 0, W - 1)
    cx1 = jnp.clip(xi1, 0, W - 1)
    cy0 = jnp.clip(yi0, 0, H - 1)
    cy1 = jnp.clip(yi1, 0, H - 1)
    expm = jnp.exp(metric_ref[0, 0])

    def vx(xi):
        return (xi >= 0) & (xi < W)

    def vy(yi):
        return (yi >= 0) & (yi < H)

    packed_ref[0] = (
        (cy0 * W + cx0) * 4 + (cy1 - cy0) * 2 + (cx1 - cx0)
        + (vx(xi0).astype(jnp.int32) << 20)
        + (vx(xi1).astype(jnp.int32) << 21)
    )
    zero = jnp.zeros_like(fx)
    tx_ref[0] = fx - x0f
    wn_ref[0] = jnp.where(vy(yi0), (y1f - fy) * expm, zero)
    ws_ref[0] = jnp.where(vy(yi1), (fy - y0f) * expm, zero)


def _prep(flow, metric, *, B, H, W, Hb=64):
    io_spec = pl.BlockSpec((1, Hb, W), lambda b, h: (b, h, 0))
    return pl.pallas_call(
        functools.partial(_prep_body, Hb=Hb, H=H, W=W),
        grid=(B, H // Hb),
        in_specs=[
            pl.BlockSpec((1, 2, Hb, W), lambda b, h: (b, 0, h, 0)),
            pl.BlockSpec((1, 1, Hb, W), lambda b, h: (b, 0, h, 0)),
        ],
        out_specs=[io_spec] * 4,
        out_shape=[jax.ShapeDtypeStruct((B, H, W), jnp.int32)]
        + [jax.ShapeDtypeStruct((B, H, W), jnp.float32)] * 3,
    )(flow, metric)


def _splat(x_flat, packed, tx, wn, ws, *, B, C, HW, W):
    PPT = HW // NS
    WIN = 1024
    NWIN = PPT // WIN
    ZB = 4096
    mesh = plsc.VectorSubcoreMesh(
        core_axis_name="c", subcore_axis_name="s", num_cores=NC, num_subcores=NS
    )

    @functools.partial(
        pl.kernel,
        out_type=jax.ShapeDtypeStruct((B * C * HW,), jnp.float32),
        mesh=mesh,
        scratch_types=[
            pltpu.VMEM((PPT,), jnp.int32),
            pltpu.VMEM((PPT,), jnp.float32),
            pltpu.VMEM((PPT,), jnp.float32),
            pltpu.VMEM((PPT,), jnp.float32),
            pltpu.VMEM((PPT // 2,), jnp.float32),
            pltpu.VMEM((PPT,), jnp.float32),
            [[pltpu.VMEM((WIN,), jnp.int32) for _ in range(4)]
             for _ in range(2)],
            [[pltpu.VMEM((WIN,), jnp.float32) for _ in range(4)]
             for _ in range(2)],
            pltpu.VMEM((ZB,), jnp.float32),
            pltpu.VMEM_SHARED((HW,), jnp.float32),
            [pltpu.SemaphoreType.DMA for _ in range(2)],
            pltpu.SemaphoreType.DMA,
        ],
    )
    def splat_kernel(x_hbm, pk_hbm, tx_hbm, wn_hbm, ws_hbm, out_hbm,
                     pk_v, tx_v, wn_v, ws_v, x_v, den_v, idx4_v, upd4_v,
                     zero_v, plane_sh, ssem, osem):
        sc = lax.axis_index("c")
        sid = lax.axis_index("s")
        off = sc * HW + sid * PPT
        q0 = sid * PPT
        HALF = PPT // 2
        WPH = HALF // WIN

        pltpu.sync_copy(pk_hbm.at[pl.ds(off, PPT)], pk_v)
        pltpu.sync_copy(tx_hbm.at[pl.ds(off, PPT)], tx_v)
        pltpu.sync_copy(wn_hbm.at[pl.ds(off, PPT)], wn_v)
        pltpu.sync_copy(ws_hbm.at[pl.ds(off, PPT)], ws_v)

        def zbody(i, _):
            zero_v[pl.ds(i * L, L)] = jnp.zeros((L,), jnp.float32)
            return 0

        lax.fori_loop(0, ZB // L, zbody, 0)

        def zero_my_slice():
            for r in range(PPT // ZB):
                pltpu.sync_copy(zero_v, plane_sh.at[pl.ds(q0 + r * ZB, ZB)])

        zero_my_slice()
        plsc.subcore_barrier()

        def do_windows(with_x, c):
            pend = [None, None]
            for wnd in range(NWIN):
                bi = wnd % 2
                idxb, updb = idx4_v[bi], upd4_v[bi]
                if pend[bi] is not None:
                    for d in pend[bi]:
                        d.wait()

                def body(i, _, wnd=wnd, idxb=idxb, updb=updb):
                    j = wnd * (WIN // L) + i
                    s = pl.ds(j * L, L)
                    pk = pk_v[s]
                    base = (pk >> 2) & 0x3FFFF
                    dxb = pk & 1
                    dyw = (pk & 2) << 8
                    vx0 = (pk & (1 << 20)) != 0
                    vx1 = (pk & (1 << 21)) != 0
                    tx = tx_v[s]
                    wn = wn_v[s]
                    ws = ws_v[s]
                    omt = 1.0 - tx
                    zero = jnp.zeros((L,), jnp.float32)
                    w4 = (
                        jnp.where(vx0, omt * wn, zero),
                        jnp.where(vx1, tx * wn, zero),
                        jnp.where(vx0, omt * ws, zero),
                        jnp.where(vx1, tx * ws, zero),
                    )
                    if with_x:
                        xv = x_v[pl.ds((wnd % WPH) * WIN + i * L, L)]
                    sw = pl.ds(i * L, L)
                    for k, idx in enumerate(
                            (base, base + dxb, base + dyw, base + dxb + dyw)):
                        idxb[k][sw] = idx
                        updb[k][sw] = (xv * w4[k]) if with_x else w4[k]
                    return 0

                lax.fori_loop(0, WIN // L, body, 0)
                pend[bi] = [
                    pltpu.async_copy(
                        updb[k], plane_sh.at[idxb[k]], ssem[bi], add=True
                    )
                    for k in range(4)
                ]
                if with_x and wnd == WPH - 1:
                    pltpu.sync_copy(
                        x_hbm.at[pl.ds((sc * C + c) * HW + q0 + HALF, HALF)],
                        x_v)
            for p in pend:
                if p is not None:
                    for d in p:
                        d.wait()

        do_windows(False, 0)
        plsc.subcore_barrier()
        pltpu.sync_copy(plane_sh.at[pl.ds(q0, PPT)], den_v)
        zero_my_slice()
        plsc.subcore_barrier()

        def chan_body(c, _):
            pltpu.sync_copy(x_hbm.at[pl.ds((sc * C + c) * HW + q0, HALF)], x_v)
            do_windows(True, c)
            plsc.subcore_barrier()
            odesc = None
            for half in range(2):
                hq = half * HALF
                pltpu.sync_copy(plane_sh.at[pl.ds(q0 + hq, HALF)], x_v)

                def nbody(i, _, hq=hq):
                    s = pl.ds(i * L, L)
                    x_v[s] = x_v[s] / (den_v[pl.ds(hq + i * L, L)] + 1e-7)
                    return 0

                lax.fori_loop(0, HALF // L, nbody, 0)
                if half == 0:
                    pltpu.sync_copy(
                        x_v,
                        out_hbm.at[pl.ds((sc * C + c) * HW + q0 + hq, HALF)])
                else:
                    odesc = pltpu.async_copy(
                        x_v,
                        out_hbm.at[pl.ds((sc * C + c) * HW + q0 + hq, HALF)],
                        osem)
            zero_my_slice()
            odesc.wait()
            plsc.subcore_barrier()
            return 0

        lax.fori_loop(0, C, chan_body, 0)

    return splat_kernel(x_flat, packed, tx, wn, ws)


def kernel(x, flow, metric):
    B, C, H, W = x.shape
    HW = H * W
    N = B * HW
    pk, tx, wn, ws = _prep(flow, metric, B=B, H=H, W=W)
    out = _splat(
        x.reshape(B * C * HW),
        pk.reshape(N), tx.reshape(N), wn.reshape(N), ws.reshape(N),
        B=B, C=C, HW=HW, W=W,
    )
    return out.reshape(B, C, H, W)
